# baseline (speedup 1.0000x reference)
import functools

import jax
import jax.numpy as jnp
from jax import lax
from jax.experimental import pallas as pl
from jax.experimental.pallas import tpu as pltpu
from jax.experimental.pallas import tpu_sc as plsc

VOCAB = 1000000
EMB_DIM = 64
BATCH = 16384
HIST = 200

NC, NS = 2, 16
NW = NC * NS
B_TOTAL = BATCH * HIST
TILE_B = 128
NBB = BATCH // TILE_B
N_UNITS = B_TOTAL // TILE_B
U_PER_W = N_UNITS // NW
NBUF = 4
IB = 40
N_IBLK = U_PER_W // IB
N_GROUPS = U_PER_W // NBUF

_MESH = plsc.VectorSubcoreMesh(
    core_axis_name="c", subcore_axis_name="s", num_cores=NC, num_subcores=NS
)


@functools.partial(
    pl.kernel,
    out_type=jax.ShapeDtypeStruct((HIST, 8, NBB, 8, TILE_B), jnp.float32),
    mesh=_MESH,
    scratch_types=(
        [
            pltpu.VMEM((2, IB * TILE_B), jnp.int32),
            pltpu.VMEM((NBUF, TILE_B, EMB_DIM), jnp.float32),
            pltpu.VMEM((NBUF, EMB_DIM, TILE_B), jnp.float32),
        ]
        + [pltpu.SemaphoreType.DMA] * NBUF
        + [pltpu.SemaphoreType.DMA] * NBUF
        + [pltpu.SemaphoreType.DMA]
    ),
    compiler_params=pltpu.CompilerParams(use_tc_tiling_on_sc=False,
                                        needs_layout_passes=False),
)
def _gather_kernel(idx_hbm, table_hbm, out_hbm, idx_v, g_v, t_v, *sems):
    gsem = sems[:NBUF]
    wsem = sems[NBUF : 2 * NBUF]
    isem = sems[2 * NBUF]
    wid = lax.axis_index("s") * NC + lax.axis_index("c")
    ubase = wid * U_PER_W
    ibase = ubase * TILE_B

    lane = lax.iota(jnp.int32, 16)
    evs = [lane + 16 * j for j in range(4)]

    pltpu.sync_copy(idx_hbm.at[pl.ds(ibase, IB * TILE_B)], idx_v.at[0])
    for b in range(NBUF):
        pltpu.async_copy(
            table_hbm.at[idx_v.at[0, pl.ds(b * TILE_B, TILE_B)]],
            g_v.at[b],
            gsem[b],
        )

    def group_step(g, carry):
        u0 = g * NBUF
        blk = u0 // IB
        pos = lax.rem(u0, IB)

        @pl.when((pos == 0) & (blk + 1 < N_IBLK))
        def _():
            pltpu.async_copy(
                idx_hbm.at[pl.ds(ibase + (blk + 1) * IB * TILE_B, IB * TILE_B)],
                idx_v.at[lax.rem(blk + 1, 2)],
                isem,
            )

        @pl.when((pos == IB - NBUF) & (blk + 1 < N_IBLK))
        def _():
            pltpu.make_async_copy(
                idx_hbm.at[pl.ds(ibase, IB * TILE_B)],
                idx_v.at[lax.rem(blk + 1, 2)],
                isem,
            ).wait()

        for b in range(NBUF):
            u = u0 + b
            ug = ubase + u

            pltpu.make_async_copy(
                table_hbm.at[
                    idx_v.at[lax.rem(u // IB, 2),
                             pl.ds(lax.rem(u, IB) * TILE_B, TILE_B)]
                ],
                g_v.at[b],
                gsem[b],
            ).wait()

            @pl.when(u >= NBUF)
            def _(b=b):
                for eb in range(8):
                    pltpu.make_async_copy(
                        t_v.at[b, pl.ds(eb * 8, 8)],
                        out_hbm.at[0, eb, 0],
                        wsem[b],
                    ).wait()

            def tr_step(rr, c2, b=b):
                r = rr * 4
                for dr in range(4):
                    row = r + dr
                    rowv = jnp.full((16,), row, jnp.int32)
                    for j in range(4):
                        v = g_v[b, row, pl.ds(j * 16, 16)]
                        plsc.store_scatter(t_v.at[b], [evs[j], rowv], v)
                return c2

            lax.fori_loop(0, TILE_B // 4, tr_step, 0)

            h = ug // NBB
            bb = lax.rem(ug, NBB)
            for eb in range(8):
                pltpu.async_copy(
                    t_v.at[b, pl.ds(eb * 8, 8)],
                    out_hbm.at[h, eb, bb],
                    wsem[b],
                )

            un = u + NBUF

            @pl.when(un < U_PER_W)
            def _(b=b, un=un):
                pltpu.async_copy(
                    table_hbm.at[
                        idx_v.at[lax.rem(un // IB, 2),
                                 pl.ds(lax.rem(un, IB) * TILE_B, TILE_B)]
                    ],
                    g_v.at[b],
                    gsem[b],
                )
        return carry

    lax.fori_loop(0, N_GROUPS, group_step, 0)

    for b in range(NBUF):
        for eb in range(8):
            pltpu.make_async_copy(
                t_v.at[b, pl.ds(eb * 8, 8)], out_hbm.at[0, eb, 0], wsem[b]
            ).wait()


def kernel(x, table):
    xt = jnp.transpose(x).reshape(-1).astype(jnp.int32)
    out5 = _gather_kernel(xt, table)
    return jnp.transpose(out5, (2, 4, 0, 1, 3)).reshape(BATCH, HIST, EMB_DIM)


# speedup vs baseline: 5.7285x; 5.7285x over previous
import functools

import jax
import jax.numpy as jnp
from jax import lax
from jax.experimental import pallas as pl
from jax.experimental.pallas import tpu as pltpu
from jax.experimental.pallas import tpu_sc as plsc

VOCAB = 1000000
EMB_DIM = 64
BATCH = 16384
HIST = 200

NC, NS = 2, 16
NW = NC * NS
B_TOTAL = BATCH * HIST
TILE_B = 128
NBB = BATCH // TILE_B
N_UNITS = B_TOTAL // TILE_B
U_PER_W = N_UNITS // NW
NBUF = 5
IB = 40
N_IBLK = U_PER_W // IB
N_GROUPS = U_PER_W // NBUF

_MESH = plsc.VectorSubcoreMesh(
    core_axis_name="c", subcore_axis_name="s", num_cores=NC, num_subcores=NS
)


@functools.partial(
    pl.kernel,
    out_type=jax.ShapeDtypeStruct((HIST, 8, NBB, 8, TILE_B), jnp.float32),
    mesh=_MESH,
    scratch_types=(
        [
            pltpu.VMEM((2, IB * TILE_B), jnp.int32),
            pltpu.VMEM((NBUF, TILE_B, EMB_DIM), jnp.float32),
            pltpu.VMEM((NBUF, EMB_DIM, TILE_B), jnp.float32),
        ]
        + [pltpu.SemaphoreType.DMA] * NBUF
        + [pltpu.SemaphoreType.DMA] * NBUF
        + [pltpu.SemaphoreType.DMA]
    ),
    compiler_params=pltpu.CompilerParams(use_tc_tiling_on_sc=False,
                                        needs_layout_passes=False),
)
def _gather_kernel(idx_hbm, table_hbm, out_hbm, idx_v, g_v, t_v, *sems):
    gsem = sems[:NBUF]
    wsem = sems[NBUF : 2 * NBUF]
    isem = sems[2 * NBUF]
    wid = lax.axis_index("s") * NC + lax.axis_index("c")
    ubase = wid * U_PER_W
    ibase = ubase * TILE_B

    lane = lax.iota(jnp.int32, 16)
    evs = [lane + 16 * j for j in range(4)]

    pltpu.sync_copy(idx_hbm.at[pl.ds(ibase, IB * TILE_B)], idx_v.at[0])
    for b in range(NBUF):
        pltpu.async_copy(
            table_hbm.at[idx_v.at[0, pl.ds(b * TILE_B, TILE_B)]],
            g_v.at[b],
            gsem[b],
        )

    def group_step(g, carry):
        u0 = g * NBUF
        blk = u0 // IB
        pos = lax.rem(u0, IB)

        @pl.when((pos == 0) & (blk + 1 < N_IBLK))
        def _():
            pltpu.async_copy(
                idx_hbm.at[pl.ds(ibase + (blk + 1) * IB * TILE_B, IB * TILE_B)],
                idx_v.at[lax.rem(blk + 1, 2)],
                isem,
            )

        @pl.when((pos == IB - NBUF) & (blk + 1 < N_IBLK))
        def _():
            pltpu.make_async_copy(
                idx_hbm.at[pl.ds(ibase, IB * TILE_B)],
                idx_v.at[lax.rem(blk + 1, 2)],
                isem,
            ).wait()

        for b in range(NBUF):
            u = u0 + b
            ug = ubase + u

            pltpu.make_async_copy(
                table_hbm.at[
                    idx_v.at[lax.rem(u // IB, 2),
                             pl.ds(lax.rem(u, IB) * TILE_B, TILE_B)]
                ],
                g_v.at[b],
                gsem[b],
            ).wait()

            @pl.when(u >= NBUF)
            def _(b=b):
                for eb in range(8):
                    pltpu.make_async_copy(
                        t_v.at[b, pl.ds(eb * 8, 8)],
                        out_hbm.at[0, eb, 0],
                        wsem[b],
                    ).wait()

            def tr_step(rr, m, b=b):
                m2 = jnp.bitwise_and(m + 1, TILE_B - 1)
                vs = []
                for mm in (m, m2):
                    for j in range(4):
                        vs.append(plsc.load_gather(g_v.at[b], [mm, evs[j]]))
                k = 0
                for mm in (m, m2):
                    for j in range(4):
                        plsc.store_scatter(t_v.at[b], [evs[j], mm], vs[k])
                        k += 1
                return jnp.bitwise_and(m + 2, TILE_B - 1)

            lax.fori_loop(0, TILE_B // 2, tr_step, lane)

            h = ug // NBB
            bb = lax.rem(ug, NBB)
            for eb in range(8):
                pltpu.async_copy(
                    t_v.at[b, pl.ds(eb * 8, 8)],
                    out_hbm.at[h, eb, bb],
                    wsem[b],
                )

            un = u + NBUF

            @pl.when(un < U_PER_W)
            def _(b=b, un=un):
                pltpu.async_copy(
                    table_hbm.at[
                        idx_v.at[lax.rem(un // IB, 2),
                                 pl.ds(lax.rem(un, IB) * TILE_B, TILE_B)]
                    ],
                    g_v.at[b],
                    gsem[b],
                )
        return carry

    lax.fori_loop(0, N_GROUPS, group_step, 0)

    for b in range(NBUF):
        for eb in range(8):
            pltpu.make_async_copy(
                t_v.at[b, pl.ds(eb * 8, 8)], out_hbm.at[0, eb, 0], wsem[b]
            ).wait()


TP_NFULL = VOCAB // TILE_B
TP_REM = VOCAB - TP_NFULL * TILE_B
TP_PER_W = 245
TP_NBUF = 7


@functools.partial(
    pl.kernel,
    out_type=jax.ShapeDtypeStruct((VOCAB * EMB_DIM,), jnp.float32),
    mesh=_MESH,
    scratch_types=(
        [
            pltpu.VMEM((TP_NBUF, EMB_DIM, TILE_B), jnp.float32),
            pltpu.VMEM((TP_NBUF * TILE_B * EMB_DIM,), jnp.float32),
        ]
        + [pltpu.SemaphoreType.DMA] * TP_NBUF
        + [pltpu.SemaphoreType.DMA] * TP_NBUF
    ),
    compiler_params=pltpu.CompilerParams(use_tc_tiling_on_sc=True,
                                        needs_layout_passes=False),
)
def _prep_kernel(tabt_hbm, out_hbm, g2_v, t2_v, *sems):
    rsem = sems[:TP_NBUF]
    wsem = sems[TP_NBUF : 2 * TP_NBUF]
    wid = lax.axis_index("s") * NC + lax.axis_index("c")
    start = (wid * (TP_NFULL - TP_PER_W)) // (NW - 1)

    lane = lax.iota(jnp.int32, 16)
    evs = [lane + 16 * j for j in range(4)]

    def rd(blk, b):
        pltpu.async_copy(
            tabt_hbm.at[:, pl.ds(blk * TILE_B, TILE_B)], g2_v.at[b], rsem[b]
        )

    def rd_wait(b):
        pltpu.make_async_copy(
            tabt_hbm.at[:, pl.ds(0, TILE_B)], g2_v.at[b], rsem[b]
        ).wait()

    for b in range(TP_NBUF):
        rd(start + b, b)

    def blk_group(g, carry):
        for b in range(TP_NBUF):
            t = g * TP_NBUF + b
            blk = start + t
            rd_wait(b)

            t2s = t2_v.at[pl.ds(b * TILE_B * EMB_DIM, TILE_B * EMB_DIM)]

            @pl.when(t >= TP_NBUF)
            def _(b=b, t2s=t2s):
                pltpu.make_async_copy(
                    t2s, out_hbm.at[pl.ds(0, TILE_B * EMB_DIM)],
                    wsem[b],
                ).wait()

            def tr_step(rr, m, b=b):
                m2 = jnp.bitwise_and(m + 1, TILE_B - 1)
                vs = []
                for mm in (m, m2):
                    for j in range(4):
                        vs.append(plsc.load_gather(g2_v.at[b], [evs[j], mm]))
                k = 0
                for mm in (m, m2):
                    for j in range(4):
                        plsc.store_scatter(
                            t2s, [mm * EMB_DIM + evs[j]], vs[k]
                        )
                        k += 1
                return jnp.bitwise_and(m + 2, TILE_B - 1)

            lax.fori_loop(0, TILE_B // 2, tr_step, lane)

            pltpu.async_copy(
                t2s,
                out_hbm.at[pl.ds(blk * TILE_B * EMB_DIM, TILE_B * EMB_DIM)],
                wsem[b],
            )

            @pl.when(t + TP_NBUF < TP_PER_W)
            def _(b=b, t=t):
                rd(start + t + TP_NBUF, b)
        return carry

    lax.fori_loop(0, TP_PER_W // TP_NBUF, blk_group, 0)

    for b in range(TP_NBUF):
        pltpu.make_async_copy(
            t2_v.at[pl.ds(b * TILE_B * EMB_DIM, TILE_B * EMB_DIM)],
            out_hbm.at[pl.ds(0, TILE_B * EMB_DIM)],
            wsem[b],
        ).wait()


def kernel(x, table):
    xt = jnp.transpose(x).reshape(-1).astype(jnp.int32)
    tflat = _prep_kernel(jnp.transpose(table))
    tflat = lax.dynamic_update_slice(
        tflat,
        table[VOCAB - TP_REM :, :].reshape(-1),
        ((VOCAB - TP_REM) * EMB_DIM,),
    )
    out5 = _gather_kernel(xt, tflat.reshape(VOCAB, EMB_DIM))
    return jnp.transpose(out5, (2, 4, 0, 1, 3)).reshape(BATCH, HIST, EMB_DIM)
